# baseline (speedup 1.0000x reference)
import functools

import jax
import jax.numpy as jnp
from jax import lax
from jax.experimental import pallas as pl
from jax.experimental.pallas import tpu as pltpu
from jax.experimental.pallas import tpu_sc as plsc

NS = 16
NC = 2
LANES = 16
D = 64
NREG = D // LANES
CH = 1024
GB = 256
PEND = GB + 32
ZROWS = 64
MAX_BS = 20480


def _pick_blocks(n_dst):
    for nb in (2, 4, 6, 8, 10, 12):
        bs = -(-n_dst // nb)
        bs = -(-bs // 4096) * 4096
        if bs <= MAX_BS:
            return nb, bs
    raise ValueError(f"n_dst {n_dst} too large")


def _pad_edges(dst, src, vals, n_dst):
    e = dst.shape[0]
    e_pad = -(-e // (2 * NS * CH)) * (2 * NS * CH)
    if e_pad != e:
        pad = e_pad - e
        pdst = (jnp.arange(pad, dtype=jnp.int32) * 97) % n_dst
        psrc = (jnp.arange(pad, dtype=jnp.int32) * 89) % n_dst
        dst = jnp.concatenate([dst, pdst])
        src = jnp.concatenate([src, psrc])
        vals = jnp.concatenate([vals, jnp.zeros((pad,), jnp.float32)])
    return dst, src, vals, e_pad


@functools.partial(jax.jit, static_argnames=("n_dst",))
def _sc_bucketize(dst, src, vals, n_dst):
    dst, src, vals, e_pad = _pad_edges(dst, src, vals, n_dst)
    nb, bs = _pick_blocks(n_dst)
    nbh = nb // NC
    per = e_pad // NS
    rg = 16 + per + GB
    nw = NC * NS
    mesh = plsc.VectorSubcoreMesh(core_axis_name="c", subcore_axis_name="s")

    @functools.partial(
        pl.kernel,
        out_type=(jax.ShapeDtypeStruct((nw, nbh, rg), jnp.int32),
                  jax.ShapeDtypeStruct((nw, nbh, rg), jnp.int32),
                  jax.ShapeDtypeStruct((nw, nbh, rg), jnp.float32)),
        mesh=mesh,
        compiler_params=pltpu.CompilerParams(needs_layout_passes=False,
                                             use_tc_tiling_on_sc=False),
        scratch_types=[
            pltpu.VMEM((2, CH), jnp.int32),
            pltpu.VMEM((2, CH), jnp.int32),
            pltpu.VMEM((2, CH), jnp.float32),
            pltpu.VMEM((nbh, PEND), jnp.int32),
            pltpu.VMEM((nbh, PEND), jnp.int32),
            pltpu.VMEM((nbh, PEND), jnp.float32),
            pltpu.VMEM((16,), jnp.int32),
            pltpu.SemaphoreType.DMA,
            pltpu.SemaphoreType.DMA,
        ],
    )
    def bkt(dst_hbm, src_hbm, val_hbm, bd_hbm, bsr_hbm, bv_hbm,
            dch, sch, vch, pdst, psrc, pval, hdr, sem0, sem1):
        c = lax.axis_index("c")
        w = lax.axis_index("s")
        wid = c * NS + w
        zi = jnp.zeros((LANES,), jnp.int32)
        zf = jnp.zeros((LANES,), jnp.float32)
        sems = (sem0, sem1)

        @pl.loop(0, PEND // LANES)
        def _(i):
            sl = pl.ds(i * LANES, LANES)
            for k in range(nbh):
                pdst.at[k][sl] = zi
                psrc.at[k][sl] = zi
                pval.at[k][sl] = zf

        def start_load(ci, b):
            off = w * per + ci * CH
            pltpu.async_copy(dst_hbm.at[pl.ds(off, CH)], dch.at[b], sems[b])
            pltpu.async_copy(src_hbm.at[pl.ds(off, CH)], sch.at[b], sems[b])
            pltpu.async_copy(val_hbm.at[pl.ds(off, CH)], vch.at[b], sems[b])

        def wait_load(b):
            off = w * per
            pltpu.make_async_copy(dst_hbm.at[pl.ds(off, CH)], dch.at[b],
                                  sems[b]).wait()
            pltpu.make_async_copy(src_hbm.at[pl.ds(off, CH)], sch.at[b],
                                  sems[b]).wait()
            pltpu.make_async_copy(val_hbm.at[pl.ds(off, CH)], vch.at[b],
                                  sems[b]).wait()

        def flush(k, blk):
            off = pl.ds(16 + blk * GB, GB)
            pltpu.sync_copy(pdst.at[k, pl.ds(0, GB)], bd_hbm.at[wid, k, off])
            pltpu.sync_copy(psrc.at[k, pl.ds(0, GB)], bsr_hbm.at[wid, k, off])
            pltpu.sync_copy(pval.at[k, pl.ds(0, GB)], bv_hbm.at[wid, k, off])

        def compact(b, st):
            db, sb, vb = dch.at[b], sch.at[b], vch.at[b]

            def vec_body(i, st):
                sl = pl.ds(i * LANES, LANES)
                d = db[sl]
                s = sb[sl]
                v = vb[sl]
                nst = []
                for k in range(nbh):
                    p, blk = st[2 * k], st[2 * k + 1]
                    lo = (k * NC + c) * bs
                    m = (d >= lo) & (d < lo + bs)
                    plsc.store_compressed(pdst.at[k, pl.ds(p, LANES)],
                                          d - lo, mask=m)
                    plsc.store_compressed(psrc.at[k, pl.ds(p, LANES)],
                                          s, mask=m)
                    plsc.store_compressed(pval.at[k, pl.ds(p, LANES)],
                                          v, mask=m)
                    p = p + plsc.all_reduce_population_count(m)[0]
                    full = p >= GB

                    @pl.when(full)
                    def _(k=k, blk=blk):
                        flush(k, blk)
                        rd = pdst[k, pl.ds(GB, LANES)]
                        rs = psrc[k, pl.ds(GB, LANES)]
                        rv = pval[k, pl.ds(GB, LANES)]
                        pdst.at[k][pl.ds(0, LANES)] = rd
                        psrc.at[k][pl.ds(0, LANES)] = rs
                        pval.at[k][pl.ds(0, LANES)] = rv

                    nst.append(jnp.where(full, p - GB, p))
                    nst.append(jnp.where(full, blk + 1, blk))
                return tuple(nst)

            return lax.fori_loop(0, CH // LANES, vec_body, st)

        start_load(0, 0)

        def pair_body(gp, st):
            ci = gp * 2
            start_load(ci + 1, 1)
            wait_load(0)
            st = compact(0, st)

            @pl.when(ci + 2 < nchunks)
            def _():
                start_load(ci + 2, 0)

            wait_load(1)
            return compact(1, st)

        nchunks = per // CH
        zero = jnp.int32(0)
        st = lax.fori_loop(0, nchunks // 2, pair_body, (zero,) * (2 * nbh))

        iot = lax.broadcasted_iota(jnp.int32, (LANES,), 0)
        for k in range(nbh):
            p, blk = st[2 * k], st[2 * k + 1]
            for j in range(GB // LANES):
                sl = pl.ds(j * LANES, LANES)
                keep = (iot + j * LANES) < p
                pval.at[k][sl] = jnp.where(keep, pval[k, sl], 0.0)
            flush(k, blk)
            hdr[pl.ds(0, LANES)] = jnp.where(iot == 0, blk + 1, 0)
            pltpu.sync_copy(hdr, bd_hbm.at[wid, k, pl.ds(0, 16)])

    return bkt(dst, src, vals)


@functools.partial(jax.jit, static_argnames=("n_dst",))
def _sc_spmm_bkt(bd, bsr, bv, table, n_dst):
    nb, bs = _pick_blocks(n_dst)
    nbh = nb // NC
    rps = bs // NS
    mesh = plsc.VectorSubcoreMesh(core_axis_name="c", subcore_axis_name="s")

    @functools.partial(
        pl.kernel,
        out_type=jax.ShapeDtypeStruct((nb * bs, D), jnp.float32),
        mesh=mesh,
        compiler_params=pltpu.CompilerParams(needs_layout_passes=False,
                                             use_tc_tiling_on_sc=False),
        scratch_types=[
            pltpu.VMEM((GB,), jnp.int32),
            pltpu.VMEM((GB,), jnp.int32),
            pltpu.VMEM((GB + 16,), jnp.float32),
            pltpu.VMEM((GB,), jnp.int32),
            pltpu.VMEM((GB,), jnp.int32),
            pltpu.VMEM((GB + 16,), jnp.float32),
            pltpu.VMEM((GB, D), jnp.float32),
            pltpu.VMEM((GB, D), jnp.float32),
            pltpu.VMEM((16,), jnp.int32),
            pltpu.VMEM((ZROWS, D), jnp.float32),
            pltpu.VMEM_SHARED((bs, D), jnp.float32),
            pltpu.SemaphoreType.DMA,
            pltpu.SemaphoreType.DMA,
            pltpu.SemaphoreType.DMA,
            pltpu.SemaphoreType.DMA,
            pltpu.SemaphoreType.DMA,
            pltpu.SemaphoreType.DMA,
        ],
    )
    def spmm(bd_hbm, bsr_hbm, bv_hbm, tab_hbm, out_hbm,
             fdst0, fsrc0, fval0, fdst1, fsrc1, fval1, rows0, rows1,
             hdr, zbuf, acc, lsem0, lsem1, gsem0, gsem1, ssem0, ssem1):
        c = lax.axis_index("c")
        w = lax.axis_index("s")
        wid = c * NS + w
        zf = jnp.zeros((LANES,), jnp.float32)
        fbufs = ((fdst0, fsrc0, fval0, rows0, lsem0, gsem0, ssem0),
                 (fdst1, fsrc1, fval1, rows1, lsem1, gsem1, ssem1))

        @pl.loop(0, ZROWS)
        def _(i):
            for r in range(NREG):
                zbuf[i, pl.ds(r * LANES, LANES)] = zf

        for k in range(nbh):
            lo = (k * NC + c) * bs
            for j in range(rps // ZROWS):
                pltpu.sync_copy(zbuf, acc.at[pl.ds(w * rps + j * ZROWS,
                                                   ZROWS)])
            plsc.subcore_barrier()

            pltpu.sync_copy(bd_hbm.at[wid, k, pl.ds(0, 16)], hdr)
            nblk = hdr[pl.ds(0, LANES)][0]

            def start_loads(fb, j, nf):
                fd, fs, fv, rw, ls, gs, ss = fbufs[fb]

                @pl.when(nf >= 2)
                def _():
                    pltpu.make_async_copy(rw, acc.at[fd], ss).wait()

                off = pl.ds(16 + j * GB, GB)
                pltpu.async_copy(bd_hbm.at[wid, k, off], fd, ls)
                pltpu.async_copy(bsr_hbm.at[wid, k, off], fs, ls)
                pltpu.async_copy(bv_hbm.at[wid, k, off], fv.at[pl.ds(0, GB)],
                                 ls)

            def process(pb, nf):
                fd, fs, fv, rw, ls, gs, ss = fbufs[pb]

                @pl.when(nf >= 1)
                def _():
                    pltpu.make_async_copy(tab_hbm.at[fs], rw, gs).wait()

                    @plsc.parallel_loop(0, GB, unroll=8)
                    def _(ei):
                        v = fv[pl.ds(ei, LANES)][0]
                        for r in range(NREG):
                            sl = pl.ds(r * LANES, LANES)
                            rw[ei, sl] = rw[ei, sl] * v

                    pltpu.async_copy(rw, acc.at[fd], ss, add=True)

            def fin_gather(fb, j):
                fd, fs, fv, rw, ls, gs, ss = fbufs[fb]
                off = pl.ds(16 + j * GB, GB)
                pltpu.make_async_copy(bd_hbm.at[wid, k, off], fd, ls).wait()
                pltpu.make_async_copy(bsr_hbm.at[wid, k, off], fs, ls).wait()
                pltpu.make_async_copy(bv_hbm.at[wid, k, off],
                                      fv.at[pl.ds(0, GB)], ls).wait()
                pltpu.async_copy(tab_hbm.at[fs], rw, gs)

            def step(fb, j, nf):
                start_loads(fb, j, nf)
                process(1 - fb, nf)
                fin_gather(fb, j)

            def blk_body(j, st):
                fb, nf = st
                lax.cond(fb == 0,
                         lambda: step(0, j, nf),
                         lambda: step(1, j, nf))
                return (1 - fb, nf + 1)

            zero = jnp.int32(0)
            fb, nf = lax.fori_loop(0, nblk, blk_body, (zero, zero))

            def fin_scatter(pb, guard):
                fd, fs, fv, rw, ls, gs, ss = fbufs[pb]

                @pl.when(guard)
                def _():
                    pltpu.make_async_copy(rw, acc.at[fd], ss).wait()

            lax.cond(fb == 0,
                     lambda: (process(1, nf), fin_scatter(1, nf >= 1),
                              fin_scatter(0, nf >= 2))[0],
                     lambda: (process(0, nf), fin_scatter(0, nf >= 1),
                              fin_scatter(1, nf >= 2))[0])
            plsc.subcore_barrier()
            pltpu.sync_copy(acc.at[pl.ds(w * rps, rps)],
                            out_hbm.at[pl.ds(lo + w * rps, rps)])
            plsc.subcore_barrier()

    return spmm(bd, bsr, bv, table)


@functools.partial(jax.jit, static_argnames=("n_dst",))
def _sc_spmm(dst, src, vals, table, n_dst):
    e = dst.shape[0]
    nb, bs = _pick_blocks(n_dst)
    e_pad = -(-e // (2 * NS * CH)) * (2 * NS * CH)
    n_tab = table.shape[0]
    if e_pad != e:
        pad = e_pad - e
        pdst = (jnp.arange(pad, dtype=jnp.int32) * 97) % n_dst
        psrc = (jnp.arange(pad, dtype=jnp.int32) * 89) % n_tab
        dst = jnp.concatenate([dst, pdst])
        src = jnp.concatenate([src, psrc])
        vals = jnp.concatenate([vals, jnp.zeros((pad,), jnp.float32)])
    per = e_pad // NS
    nchunks = per // CH
    rps = bs // NS

    mesh = plsc.VectorSubcoreMesh(core_axis_name="c", subcore_axis_name="s")

    @functools.partial(
        pl.kernel,
        out_type=jax.ShapeDtypeStruct((nb * bs, D), jnp.float32),
        mesh=mesh,
        compiler_params=pltpu.CompilerParams(needs_layout_passes=False,
                                             use_tc_tiling_on_sc=False),
        scratch_types=[
            pltpu.VMEM((2, CH), jnp.int32),
            pltpu.VMEM((2, CH), jnp.int32),
            pltpu.VMEM((2, CH), jnp.float32),
            pltpu.VMEM((PEND,), jnp.int32),
            pltpu.VMEM((PEND,), jnp.int32),
            pltpu.VMEM((PEND,), jnp.float32),
            pltpu.VMEM((GB,), jnp.int32),
            pltpu.VMEM((GB,), jnp.int32),
            pltpu.VMEM((GB,), jnp.float32),
            pltpu.VMEM((GB,), jnp.int32),
            pltpu.VMEM((GB,), jnp.int32),
            pltpu.VMEM((GB,), jnp.float32),
            pltpu.VMEM((GB, D), jnp.float32),
            pltpu.VMEM((GB, D), jnp.float32),
            pltpu.VMEM((ZROWS, D), jnp.float32),
            pltpu.VMEM_SHARED((bs, D), jnp.float32),
            pltpu.SemaphoreType.DMA,
            pltpu.SemaphoreType.DMA,
            pltpu.SemaphoreType.DMA,
            pltpu.SemaphoreType.DMA,
            pltpu.SemaphoreType.DMA,
            pltpu.SemaphoreType.DMA,
        ],
    )
    def spmm(dst_hbm, src_hbm, val_hbm, tab_hbm, out_hbm,
             dch, sch, vch, pdst, psrc, pval,
             fdst0, fsrc0, fval0, fdst1, fsrc1, fval1, rows0, rows1,
             zbuf, acc, sem0, sem1, gsem0, gsem1, ssem0, ssem1):
        c = lax.axis_index("c")
        w = lax.axis_index("s")
        zi = jnp.zeros((LANES,), jnp.int32)
        zf = jnp.zeros((LANES,), jnp.float32)
        sems = (sem0, sem1)

        @pl.loop(0, ZROWS)
        def _(i):
            for r in range(NREG):
                zbuf[i, pl.ds(r * LANES, LANES)] = zf

        @pl.loop(0, PEND // LANES)
        def _(i):
            pdst[pl.ds(i * LANES, LANES)] = zi
            psrc[pl.ds(i * LANES, LANES)] = zi
            pval[pl.ds(i * LANES, LANES)] = zf

        def start_load(ci, b):
            off = w * per + ci * CH
            pltpu.async_copy(dst_hbm.at[pl.ds(off, CH)], dch.at[b], sems[b])
            pltpu.async_copy(src_hbm.at[pl.ds(off, CH)], sch.at[b], sems[b])
            pltpu.async_copy(val_hbm.at[pl.ds(off, CH)], vch.at[b], sems[b])

        def wait_load(b):
            off = w * per
            pltpu.make_async_copy(dst_hbm.at[pl.ds(off, CH)], dch.at[b],
                                  sems[b]).wait()
            pltpu.make_async_copy(src_hbm.at[pl.ds(off, CH)], sch.at[b],
                                  sems[b]).wait()
            pltpu.make_async_copy(val_hbm.at[pl.ds(off, CH)], vch.at[b],
                                  sems[b]).wait()

        fbufs = ((fdst0, fsrc0, fval0, rows0, gsem0, ssem0),
                 (fdst1, fsrc1, fval1, rows1, gsem1, ssem1))

        def _process(pb, nf):
            fd, fs, fv, rw, gs, ss = fbufs[pb]

            @pl.when(nf >= 1)
            def _():
                pltpu.make_async_copy(tab_hbm.at[fs], rw, gs).wait()

                @plsc.parallel_loop(0, GB, unroll=8)
                def _(ei):
                    v = fv[pl.ds(ei, LANES)][0]
                    for r in range(NREG):
                        sl = pl.ds(r * LANES, LANES)
                        rw[ei, sl] = rw[ei, sl] * v

                pltpu.async_copy(rw, acc.at[fd], ss, add=True)

        def _start(b, nf):
            fd, fs, fv, rw, gs, ss = fbufs[b]

            @pl.when(nf >= 2)
            def _():
                pltpu.make_async_copy(rw, acc.at[fd], ss).wait()

            for j in range(GB // LANES):
                sl = pl.ds(j * LANES, LANES)
                fs[sl] = psrc[sl]
                fd[sl] = pdst[sl]
                fv[sl] = pval[sl]
            pltpu.async_copy(tab_hbm.at[fs], rw, gs)

        def fire(fb, nf):
            lax.cond(fb == 0,
                     lambda: (_process(1, nf), _start(0, nf))[0],
                     lambda: (_process(0, nf), _start(1, nf))[0])

        def _fin(pb, guard):
            fd, fs, fv, rw, gs, ss = fbufs[pb]

            @pl.when(guard)
            def _():
                pltpu.make_async_copy(rw, acc.at[fd], ss).wait()

        def drain(fb, nf):
            lax.cond(fb == 0,
                     lambda: (_process(1, nf), _fin(1, nf >= 1),
                              _fin(0, nf >= 2))[0],
                     lambda: (_process(0, nf), _fin(0, nf >= 1),
                              _fin(1, nf >= 2))[0])

        def compact(b, lo, st):
            db, sb, vb = dch.at[b], sch.at[b], vch.at[b]

            def vec_body(i, st):
                p, fb, nf = st
                sl = pl.ds(i * LANES, LANES)
                d = db[sl]
                m = (d >= lo) & (d < lo + bs)
                plsc.store_compressed(pdst.at[pl.ds(p, LANES)], d - lo,
                                      mask=m)
                plsc.store_compressed(psrc.at[pl.ds(p, LANES)], sb[sl],
                                      mask=m)
                plsc.store_compressed(pval.at[pl.ds(p, LANES)], vb[sl],
                                      mask=m)
                p = p + plsc.all_reduce_population_count(m)[0]
                full = p >= GB

                @pl.when(full)
                def _():
                    fire(fb, nf)
                    rd = pdst[pl.ds(GB, LANES)]
                    rs = psrc[pl.ds(GB, LANES)]
                    rv = pval[pl.ds(GB, LANES)]
                    pdst[pl.ds(0, LANES)] = rd
                    psrc[pl.ds(0, LANES)] = rs
                    pval[pl.ds(0, LANES)] = rv

                return (jnp.where(full, p - GB, p),
                        jnp.where(full, 1 - fb, fb),
                        jnp.where(full, nf + 1, nf))

            return lax.fori_loop(0, CH // LANES, vec_body, st)

        for k in range(nb // NC):
            b = k * NC + c
            lo = b * bs
            for j in range(rps // ZROWS):
                pltpu.sync_copy(zbuf, acc.at[pl.ds(w * rps + j * ZROWS,
                                                   ZROWS)])
            plsc.subcore_barrier()

            start_load(0, 0)

            def pair_body(gp, st):
                ci = gp * 2
                start_load(ci + 1, 1)
                wait_load(0)
                st = compact(0, lo, st)

                @pl.when(ci + 2 < nchunks)
                def _():
                    start_load(ci + 2, 0)

                wait_load(1)
                return compact(1, lo, st)

            zero = jnp.int32(0)
            ptr, fb, nf = lax.fori_loop(0, nchunks // 2, pair_body,
                                        (zero, zero, zero))

            iot = lax.broadcasted_iota(jnp.int32, (LANES,), 0)
            for j in range(GB // LANES):
                sl = pl.ds(j * LANES, LANES)
                keep = (iot + j * LANES) < ptr
                pval[sl] = jnp.where(keep, pval[sl], 0.0)
            fire(fb, nf)
            drain(1 - fb, nf + 1)
            plsc.subcore_barrier()
            pltpu.sync_copy(acc.at[pl.ds(w * rps, rps)],
                            out_hbm.at[pl.ds(lo + w * rps, rps)])
            plsc.subcore_barrier()

    return spmm(dst, src, vals, table)


def _tc_norm(x):
    r = 1024
    assert x.shape[0] % r == 0

    def body(x_ref, o_ref):
        xb = x_ref[...]
        n = jnp.sqrt(jnp.sum(xb * xb, axis=1, keepdims=True))
        o_ref[...] = xb / jnp.maximum(n, 1e-12)

    return pl.pallas_call(
        body,
        out_shape=jax.ShapeDtypeStruct(x.shape, x.dtype),
        grid=(x.shape[0] // r,),
        in_specs=[pl.BlockSpec((r, D), lambda i: (i, 0))],
        out_specs=pl.BlockSpec((r, D), lambda i: (i, 0)),
    )(x)


def _tc_combine(f0, n1, f2, w3):
    r = 1024
    assert f0.shape[0] % r == 0

    def body(w_ref, f0_ref, n1_ref, f2_ref, o_ref):
        xb = f2_ref[...]
        n = jnp.sqrt(jnp.sum(xb * xb, axis=1, keepdims=True))
        n2 = xb / jnp.maximum(n, 1e-12)
        o_ref[...] = (w_ref[0] * f0_ref[...] + w_ref[1] * n1_ref[...]
                      + w_ref[2] * n2)

    return pl.pallas_call(
        body,
        out_shape=jax.ShapeDtypeStruct(f0.shape, f0.dtype),
        grid=(f0.shape[0] // r,),
        in_specs=[
            pl.BlockSpec(memory_space=pltpu.SMEM),
            pl.BlockSpec((r, D), lambda i: (i, 0)),
            pl.BlockSpec((r, D), lambda i: (i, 0)),
            pl.BlockSpec((r, D), lambda i: (i, 0)),
        ],
        out_specs=pl.BlockSpec((r, D), lambda i: (i, 0)),
    )(w3, f0, n1, f2)


def _tc_fuse(a, b, cc, w3):
    r = 1000
    assert a.shape[0] % r == 0

    def body(w_ref, a_ref, b_ref, c_ref, o_ref):
        o_ref[...] = (w_ref[0] * a_ref[...] + w_ref[1] * b_ref[...]
                      + w_ref[2] * c_ref[...])

    return pl.pallas_call(
        body,
        out_shape=jax.ShapeDtypeStruct(a.shape, a.dtype),
        grid=(a.shape[0] // r,),
        in_specs=[
            pl.BlockSpec(memory_space=pltpu.SMEM),
            pl.BlockSpec((r, D), lambda i: (i, 0)),
            pl.BlockSpec((r, D), lambda i: (i, 0)),
            pl.BlockSpec((r, D), lambda i: (i, 0)),
        ],
        out_specs=pl.BlockSpec((r, D), lambda i: (i, 0)),
    )(w3, a, b, cc)


def _propagate(edges, vals, a_feat, b_feat, layer_coefs):
    na, nbb = a_feat.shape[0], b_feat.shape[0]
    n = na + nbb
    nb, bs = _pick_blocks(n)
    npad = nb * bs
    f0 = jnp.zeros((npad, D), jnp.float32)
    f0 = f0.at[:na].set(a_feat).at[na:n].set(b_feat)
    dst = edges[0].astype(jnp.int32)
    src = edges[1].astype(jnp.int32)
    bd, bsr, bv = _sc_bucketize(dst, src, vals, n_dst=n)
    f1 = _sc_spmm_bkt(bd, bsr, bv, f0, n_dst=n)
    n1 = _tc_norm(f1)
    f2 = _sc_spmm_bkt(bd, bsr, bv, f1, n_dst=n)
    w3 = layer_coefs.reshape(3).astype(jnp.float32)
    comb = _tc_combine(f0, n1, f2, w3)
    return comb, na


def kernel(ub_prop_edges, ub_prop_vals, ui_prop_edges, ui_prop_vals,
           bi_prop_edges, bi_prop_vals, bi_agg_rows, bi_agg_cols, bi_agg_vals,
           ui_agg_rows, ui_agg_cols, ui_agg_vals,
           users_feature, bundles_feature, items_feature,
           modal_coefs, UB_layer_coefs, UI_layer_coefs, BI_layer_coefs):
    nu = users_feature.shape[0]
    nbun = bundles_feature.shape[0]

    ub_comb, _ = _propagate(ub_prop_edges, ub_prop_vals,
                            users_feature, bundles_feature, UB_layer_coefs)
    ui_comb, _ = _propagate(ui_prop_edges, ui_prop_vals,
                            users_feature, items_feature, UI_layer_coefs)
    bi_comb, _ = _propagate(bi_prop_edges, bi_prop_vals,
                            bundles_feature, items_feature, BI_layer_coefs)

    ui_bundles = _sc_spmm(bi_agg_rows.astype(jnp.int32),
                          bi_agg_cols.astype(jnp.int32) + nu,
                          bi_agg_vals, ui_comb, n_dst=nbun)
    bi_users = _sc_spmm(ui_agg_rows.astype(jnp.int32),
                        ui_agg_cols.astype(jnp.int32) + nbun,
                        ui_agg_vals, bi_comb, n_dst=nu)

    mc = modal_coefs.reshape(3).astype(jnp.float32)
    users_rep = _tc_fuse(ub_comb[:nu], ui_comb[:nu], bi_users[:nu], mc)
    bundles_rep = _tc_fuse(ub_comb[nu:nu + nbun], ui_bundles[:nbun],
                           bi_comb[:nbun], mc)
    return users_rep, bundles_rep


# speedup vs baseline: 1.0030x; 1.0030x over previous
import functools

import jax
import jax.numpy as jnp
from jax import lax
from jax.experimental import pallas as pl
from jax.experimental.pallas import tpu as pltpu
from jax.experimental.pallas import tpu_sc as plsc

NS = 16
NC = 2
LANES = 16
D = 64
NREG = D // LANES
CH = 1024
GB = 256
PEND = GB + 32
ZROWS = 64
MAX_BS = 20480


def _pick_blocks(n_dst):
    for nb in (2, 4, 6, 8, 10, 12):
        bs = -(-n_dst // nb)
        bs = -(-bs // 4096) * 4096
        if bs <= MAX_BS:
            return nb, bs
    raise ValueError(f"n_dst {n_dst} too large")


def _pad_edges(dst, src, vals, n_dst):
    e = dst.shape[0]
    e_pad = -(-e // (2 * NS * CH)) * (2 * NS * CH)
    if e_pad != e:
        pad = e_pad - e
        pdst = (jnp.arange(pad, dtype=jnp.int32) * 97) % n_dst
        psrc = (jnp.arange(pad, dtype=jnp.int32) * 89) % n_dst
        dst = jnp.concatenate([dst, pdst])
        src = jnp.concatenate([src, psrc])
        vals = jnp.concatenate([vals, jnp.zeros((pad,), jnp.float32)])
    return dst, src, vals, e_pad


@functools.partial(jax.jit, static_argnames=("n_dst",))
def _sc_bucketize(dst, src, vals, n_dst):
    dst, src, vals, e_pad = _pad_edges(dst, src, vals, n_dst)
    nb, bs = _pick_blocks(n_dst)
    nbh = nb // NC
    per = e_pad // NS
    rg = 16 + per + GB
    nw = NC * NS
    mesh = plsc.VectorSubcoreMesh(core_axis_name="c", subcore_axis_name="s")

    @functools.partial(
        pl.kernel,
        out_type=(jax.ShapeDtypeStruct((nw, nbh, rg), jnp.int32),
                  jax.ShapeDtypeStruct((nw, nbh, rg), jnp.int32),
                  jax.ShapeDtypeStruct((nw, nbh, rg), jnp.float32)),
        mesh=mesh,
        compiler_params=pltpu.CompilerParams(needs_layout_passes=False,
                                             use_tc_tiling_on_sc=False),
        scratch_types=[
            pltpu.VMEM((2, CH), jnp.int32),
            pltpu.VMEM((2, CH), jnp.int32),
            pltpu.VMEM((2, CH), jnp.float32),
            pltpu.VMEM((nbh, PEND), jnp.int32),
            pltpu.VMEM((nbh, PEND), jnp.int32),
            pltpu.VMEM((nbh, PEND), jnp.float32),
            pltpu.VMEM((16,), jnp.int32),
            pltpu.SemaphoreType.DMA,
            pltpu.SemaphoreType.DMA,
        ],
    )
    def bkt(dst_hbm, src_hbm, val_hbm, bd_hbm, bsr_hbm, bv_hbm,
            dch, sch, vch, pdst, psrc, pval, hdr, sem0, sem1):
        c = lax.axis_index("c")
        w = lax.axis_index("s")
        wid = c * NS + w
        zi = jnp.zeros((LANES,), jnp.int32)
        zf = jnp.zeros((LANES,), jnp.float32)
        sems = (sem0, sem1)

        @pl.loop(0, PEND // LANES)
        def _(i):
            sl = pl.ds(i * LANES, LANES)
            for k in range(nbh):
                pdst.at[k][sl] = zi
                psrc.at[k][sl] = zi
                pval.at[k][sl] = zf

        def start_load(ci, b):
            off = w * per + ci * CH
            pltpu.async_copy(dst_hbm.at[pl.ds(off, CH)], dch.at[b], sems[b])
            pltpu.async_copy(src_hbm.at[pl.ds(off, CH)], sch.at[b], sems[b])
            pltpu.async_copy(val_hbm.at[pl.ds(off, CH)], vch.at[b], sems[b])

        def wait_load(b):
            off = w * per
            pltpu.make_async_copy(dst_hbm.at[pl.ds(off, CH)], dch.at[b],
                                  sems[b]).wait()
            pltpu.make_async_copy(src_hbm.at[pl.ds(off, CH)], sch.at[b],
                                  sems[b]).wait()
            pltpu.make_async_copy(val_hbm.at[pl.ds(off, CH)], vch.at[b],
                                  sems[b]).wait()

        def flush(k, blk):
            off = pl.ds(16 + blk * GB, GB)
            pltpu.sync_copy(pdst.at[k, pl.ds(0, GB)], bd_hbm.at[wid, k, off])
            pltpu.sync_copy(psrc.at[k, pl.ds(0, GB)], bsr_hbm.at[wid, k, off])
            pltpu.sync_copy(pval.at[k, pl.ds(0, GB)], bv_hbm.at[wid, k, off])

        def compact(b, st):
            db, sb, vb = dch.at[b], sch.at[b], vch.at[b]

            def vec_body(i, st):
                sl = pl.ds(i * LANES, LANES)
                d = db[sl]
                s = sb[sl]
                v = vb[sl]
                nst = []
                for k in range(nbh):
                    p, blk = st[2 * k], st[2 * k + 1]
                    lo = (k * NC + c) * bs
                    m = (d >= lo) & (d < lo + bs)
                    plsc.store_compressed(pdst.at[k, pl.ds(p, LANES)],
                                          d - lo, mask=m)
                    plsc.store_compressed(psrc.at[k, pl.ds(p, LANES)],
                                          s, mask=m)
                    plsc.store_compressed(pval.at[k, pl.ds(p, LANES)],
                                          v, mask=m)
                    p = p + plsc.all_reduce_population_count(m)[0]
                    full = p >= GB

                    @pl.when(full)
                    def _(k=k, blk=blk):
                        flush(k, blk)
                        rd = pdst[k, pl.ds(GB, LANES)]
                        rs = psrc[k, pl.ds(GB, LANES)]
                        rv = pval[k, pl.ds(GB, LANES)]
                        pdst.at[k][pl.ds(0, LANES)] = rd
                        psrc.at[k][pl.ds(0, LANES)] = rs
                        pval.at[k][pl.ds(0, LANES)] = rv

                    nst.append(jnp.where(full, p - GB, p))
                    nst.append(jnp.where(full, blk + 1, blk))
                return tuple(nst)

            return lax.fori_loop(0, CH // LANES, vec_body, st)

        start_load(0, 0)

        def pair_body(gp, st):
            ci = gp * 2
            start_load(ci + 1, 1)
            wait_load(0)
            st = compact(0, st)

            @pl.when(ci + 2 < nchunks)
            def _():
                start_load(ci + 2, 0)

            wait_load(1)
            return compact(1, st)

        nchunks = per // CH
        zero = jnp.int32(0)
        st = lax.fori_loop(0, nchunks // 2, pair_body, (zero,) * (2 * nbh))

        iot = lax.broadcasted_iota(jnp.int32, (LANES,), 0)
        for k in range(nbh):
            p, blk = st[2 * k], st[2 * k + 1]
            for j in range(GB // LANES):
                sl = pl.ds(j * LANES, LANES)
                keep = (iot + j * LANES) < p
                pval.at[k][sl] = jnp.where(keep, pval[k, sl], 0.0)
            flush(k, blk)
            hdr[pl.ds(0, LANES)] = jnp.where(iot == 0, blk + 1, 0)
            pltpu.sync_copy(hdr, bd_hbm.at[wid, k, pl.ds(0, 16)])

    return bkt(dst, src, vals)


@functools.partial(jax.jit, static_argnames=("n_dst",))
def _sc_spmm_bkt(bd, bsr, bv, table, n_dst):
    nb, bs = _pick_blocks(n_dst)
    nbh = nb // NC
    rps = bs // NS
    mesh = plsc.VectorSubcoreMesh(core_axis_name="c", subcore_axis_name="s")

    @functools.partial(
        pl.kernel,
        out_type=jax.ShapeDtypeStruct((nb * bs, D), jnp.float32),
        mesh=mesh,
        compiler_params=pltpu.CompilerParams(needs_layout_passes=False,
                                             use_tc_tiling_on_sc=False),
        scratch_types=[
            pltpu.VMEM((GB,), jnp.int32),
            pltpu.VMEM((GB,), jnp.int32),
            pltpu.VMEM((GB + 16,), jnp.float32),
            pltpu.VMEM((GB,), jnp.int32),
            pltpu.VMEM((GB,), jnp.int32),
            pltpu.VMEM((GB + 16,), jnp.float32),
            pltpu.VMEM((GB, D), jnp.float32),
            pltpu.VMEM((GB, D), jnp.float32),
            pltpu.VMEM((16,), jnp.int32),
            pltpu.VMEM((ZROWS, D), jnp.float32),
            pltpu.VMEM_SHARED((bs, D), jnp.float32),
            pltpu.SemaphoreType.DMA,
            pltpu.SemaphoreType.DMA,
            pltpu.SemaphoreType.DMA,
            pltpu.SemaphoreType.DMA,
            pltpu.SemaphoreType.DMA,
            pltpu.SemaphoreType.DMA,
        ],
    )
    def spmm(bd_hbm, bsr_hbm, bv_hbm, tab_hbm, out_hbm,
             fdst0, fsrc0, fval0, fdst1, fsrc1, fval1, rows0, rows1,
             hdr, zbuf, acc, lsem0, lsem1, gsem0, gsem1, ssem0, ssem1):
        c = lax.axis_index("c")
        w = lax.axis_index("s")
        wid = c * NS + w
        zf = jnp.zeros((LANES,), jnp.float32)
        fbufs = ((fdst0, fsrc0, fval0, rows0, lsem0, gsem0, ssem0),
                 (fdst1, fsrc1, fval1, rows1, lsem1, gsem1, ssem1))

        @pl.loop(0, ZROWS)
        def _(i):
            for r in range(NREG):
                zbuf[i, pl.ds(r * LANES, LANES)] = zf

        for k in range(nbh):
            lo = (k * NC + c) * bs
            for j in range(rps // ZROWS):
                pltpu.sync_copy(zbuf, acc.at[pl.ds(w * rps + j * ZROWS,
                                                   ZROWS)])
            plsc.subcore_barrier()

            pltpu.sync_copy(bd_hbm.at[wid, k, pl.ds(0, 16)], hdr)
            nblk = hdr[pl.ds(0, LANES)][0]

            def start_loads(fb, j, nf):
                fd, fs, fv, rw, ls, gs, ss = fbufs[fb]

                @pl.when(nf >= 2)
                def _():
                    pltpu.make_async_copy(rw, acc.at[fd], ss).wait()

                off = pl.ds(16 + j * GB, GB)
                pltpu.async_copy(bd_hbm.at[wid, k, off], fd, ls)
                pltpu.async_copy(bsr_hbm.at[wid, k, off], fs, ls)
                pltpu.async_copy(bv_hbm.at[wid, k, off], fv.at[pl.ds(0, GB)],
                                 ls)

            def process(pb, nf):
                fd, fs, fv, rw, ls, gs, ss = fbufs[pb]

                @pl.when(nf >= 1)
                def _():
                    pltpu.make_async_copy(tab_hbm.at[fs], rw, gs).wait()

                    @plsc.parallel_loop(0, GB, unroll=4)
                    def _(ei):
                        v = fv[pl.ds(ei, LANES)][0]
                        for r in range(NREG):
                            sl = pl.ds(r * LANES, LANES)
                            rw[ei, sl] = rw[ei, sl] * v

                    pltpu.async_copy(rw, acc.at[fd], ss, add=True)

            def fin_gather(fb, j):
                fd, fs, fv, rw, ls, gs, ss = fbufs[fb]
                off = pl.ds(16 + j * GB, GB)
                pltpu.make_async_copy(bd_hbm.at[wid, k, off], fd, ls).wait()
                pltpu.make_async_copy(bsr_hbm.at[wid, k, off], fs, ls).wait()
                pltpu.make_async_copy(bv_hbm.at[wid, k, off],
                                      fv.at[pl.ds(0, GB)], ls).wait()
                pltpu.async_copy(tab_hbm.at[fs], rw, gs)

            def step(fb, j, nf):
                start_loads(fb, j, nf)
                process(1 - fb, nf)
                fin_gather(fb, j)

            def blk_body(j, st):
                fb, nf = st
                lax.cond(fb == 0,
                         lambda: step(0, j, nf),
                         lambda: step(1, j, nf))
                return (1 - fb, nf + 1)

            zero = jnp.int32(0)
            fb, nf = lax.fori_loop(0, nblk, blk_body, (zero, zero))

            def fin_scatter(pb, guard):
                fd, fs, fv, rw, ls, gs, ss = fbufs[pb]

                @pl.when(guard)
                def _():
                    pltpu.make_async_copy(rw, acc.at[fd], ss).wait()

            lax.cond(fb == 0,
                     lambda: (process(1, nf), fin_scatter(1, nf >= 1),
                              fin_scatter(0, nf >= 2))[0],
                     lambda: (process(0, nf), fin_scatter(0, nf >= 1),
                              fin_scatter(1, nf >= 2))[0])
            plsc.subcore_barrier()
            pltpu.sync_copy(acc.at[pl.ds(w * rps, rps)],
                            out_hbm.at[pl.ds(lo + w * rps, rps)])
            plsc.subcore_barrier()

    return spmm(bd, bsr, bv, table)


@functools.partial(jax.jit, static_argnames=("n_dst",))
def _sc_spmm(dst, src, vals, table, n_dst):
    e = dst.shape[0]
    nb, bs = _pick_blocks(n_dst)
    e_pad = -(-e // (2 * NS * CH)) * (2 * NS * CH)
    n_tab = table.shape[0]
    if e_pad != e:
        pad = e_pad - e
        pdst = (jnp.arange(pad, dtype=jnp.int32) * 97) % n_dst
        psrc = (jnp.arange(pad, dtype=jnp.int32) * 89) % n_tab
        dst = jnp.concatenate([dst, pdst])
        src = jnp.concatenate([src, psrc])
        vals = jnp.concatenate([vals, jnp.zeros((pad,), jnp.float32)])
    per = e_pad // NS
    nchunks = per // CH
    rps = bs // NS

    mesh = plsc.VectorSubcoreMesh(core_axis_name="c", subcore_axis_name="s")

    @functools.partial(
        pl.kernel,
        out_type=jax.ShapeDtypeStruct((nb * bs, D), jnp.float32),
        mesh=mesh,
        compiler_params=pltpu.CompilerParams(needs_layout_passes=False,
                                             use_tc_tiling_on_sc=False),
        scratch_types=[
            pltpu.VMEM((2, CH), jnp.int32),
            pltpu.VMEM((2, CH), jnp.int32),
            pltpu.VMEM((2, CH), jnp.float32),
            pltpu.VMEM((PEND,), jnp.int32),
            pltpu.VMEM((PEND,), jnp.int32),
            pltpu.VMEM((PEND,), jnp.float32),
            pltpu.VMEM((GB,), jnp.int32),
            pltpu.VMEM((GB,), jnp.int32),
            pltpu.VMEM((GB,), jnp.float32),
            pltpu.VMEM((GB,), jnp.int32),
            pltpu.VMEM((GB,), jnp.int32),
            pltpu.VMEM((GB,), jnp.float32),
            pltpu.VMEM((GB, D), jnp.float32),
            pltpu.VMEM((GB, D), jnp.float32),
            pltpu.VMEM((ZROWS, D), jnp.float32),
            pltpu.VMEM_SHARED((bs, D), jnp.float32),
            pltpu.SemaphoreType.DMA,
            pltpu.SemaphoreType.DMA,
            pltpu.SemaphoreType.DMA,
            pltpu.SemaphoreType.DMA,
            pltpu.SemaphoreType.DMA,
            pltpu.SemaphoreType.DMA,
        ],
    )
    def spmm(dst_hbm, src_hbm, val_hbm, tab_hbm, out_hbm,
             dch, sch, vch, pdst, psrc, pval,
             fdst0, fsrc0, fval0, fdst1, fsrc1, fval1, rows0, rows1,
             zbuf, acc, sem0, sem1, gsem0, gsem1, ssem0, ssem1):
        c = lax.axis_index("c")
        w = lax.axis_index("s")
        zi = jnp.zeros((LANES,), jnp.int32)
        zf = jnp.zeros((LANES,), jnp.float32)
        sems = (sem0, sem1)

        @pl.loop(0, ZROWS)
        def _(i):
            for r in range(NREG):
                zbuf[i, pl.ds(r * LANES, LANES)] = zf

        @pl.loop(0, PEND // LANES)
        def _(i):
            pdst[pl.ds(i * LANES, LANES)] = zi
            psrc[pl.ds(i * LANES, LANES)] = zi
            pval[pl.ds(i * LANES, LANES)] = zf

        def start_load(ci, b):
            off = w * per + ci * CH
            pltpu.async_copy(dst_hbm.at[pl.ds(off, CH)], dch.at[b], sems[b])
            pltpu.async_copy(src_hbm.at[pl.ds(off, CH)], sch.at[b], sems[b])
            pltpu.async_copy(val_hbm.at[pl.ds(off, CH)], vch.at[b], sems[b])

        def wait_load(b):
            off = w * per
            pltpu.make_async_copy(dst_hbm.at[pl.ds(off, CH)], dch.at[b],
                                  sems[b]).wait()
            pltpu.make_async_copy(src_hbm.at[pl.ds(off, CH)], sch.at[b],
                                  sems[b]).wait()
            pltpu.make_async_copy(val_hbm.at[pl.ds(off, CH)], vch.at[b],
                                  sems[b]).wait()

        fbufs = ((fdst0, fsrc0, fval0, rows0, gsem0, ssem0),
                 (fdst1, fsrc1, fval1, rows1, gsem1, ssem1))

        def _process(pb, nf):
            fd, fs, fv, rw, gs, ss = fbufs[pb]

            @pl.when(nf >= 1)
            def _():
                pltpu.make_async_copy(tab_hbm.at[fs], rw, gs).wait()

                @plsc.parallel_loop(0, GB, unroll=4)
                def _(ei):
                    v = fv[pl.ds(ei, LANES)][0]
                    for r in range(NREG):
                        sl = pl.ds(r * LANES, LANES)
                        rw[ei, sl] = rw[ei, sl] * v

                pltpu.async_copy(rw, acc.at[fd], ss, add=True)

        def _start(b, nf):
            fd, fs, fv, rw, gs, ss = fbufs[b]

            @pl.when(nf >= 2)
            def _():
                pltpu.make_async_copy(rw, acc.at[fd], ss).wait()

            for j in range(GB // LANES):
                sl = pl.ds(j * LANES, LANES)
                fs[sl] = psrc[sl]
                fd[sl] = pdst[sl]
                fv[sl] = pval[sl]
            pltpu.async_copy(tab_hbm.at[fs], rw, gs)

        def fire(fb, nf):
            lax.cond(fb == 0,
                     lambda: (_process(1, nf), _start(0, nf))[0],
                     lambda: (_process(0, nf), _start(1, nf))[0])

        def _fin(pb, guard):
            fd, fs, fv, rw, gs, ss = fbufs[pb]

            @pl.when(guard)
            def _():
                pltpu.make_async_copy(rw, acc.at[fd], ss).wait()

        def drain(fb, nf):
            lax.cond(fb == 0,
                     lambda: (_process(1, nf), _fin(1, nf >= 1),
                              _fin(0, nf >= 2))[0],
                     lambda: (_process(0, nf), _fin(0, nf >= 1),
                              _fin(1, nf >= 2))[0])

        def compact(b, lo, st):
            db, sb, vb = dch.at[b], sch.at[b], vch.at[b]

            def vec_body(i, st):
                p, fb, nf = st
                sl = pl.ds(i * LANES, LANES)
                d = db[sl]
                m = (d >= lo) & (d < lo + bs)
                plsc.store_compressed(pdst.at[pl.ds(p, LANES)], d - lo,
                                      mask=m)
                plsc.store_compressed(psrc.at[pl.ds(p, LANES)], sb[sl],
                                      mask=m)
                plsc.store_compressed(pval.at[pl.ds(p, LANES)], vb[sl],
                                      mask=m)
                p = p + plsc.all_reduce_population_count(m)[0]
                full = p >= GB

                @pl.when(full)
                def _():
                    fire(fb, nf)
                    rd = pdst[pl.ds(GB, LANES)]
                    rs = psrc[pl.ds(GB, LANES)]
                    rv = pval[pl.ds(GB, LANES)]
                    pdst[pl.ds(0, LANES)] = rd
                    psrc[pl.ds(0, LANES)] = rs
                    pval[pl.ds(0, LANES)] = rv

                return (jnp.where(full, p - GB, p),
                        jnp.where(full, 1 - fb, fb),
                        jnp.where(full, nf + 1, nf))

            return lax.fori_loop(0, CH // LANES, vec_body, st)

        for k in range(nb // NC):
            b = k * NC + c
            lo = b * bs
            for j in range(rps // ZROWS):
                pltpu.sync_copy(zbuf, acc.at[pl.ds(w * rps + j * ZROWS,
                                                   ZROWS)])
            plsc.subcore_barrier()

            start_load(0, 0)

            def pair_body(gp, st):
                ci = gp * 2
                start_load(ci + 1, 1)
                wait_load(0)
                st = compact(0, lo, st)

                @pl.when(ci + 2 < nchunks)
                def _():
                    start_load(ci + 2, 0)

                wait_load(1)
                return compact(1, lo, st)

            zero = jnp.int32(0)
            ptr, fb, nf = lax.fori_loop(0, nchunks // 2, pair_body,
                                        (zero, zero, zero))

            iot = lax.broadcasted_iota(jnp.int32, (LANES,), 0)
            for j in range(GB // LANES):
                sl = pl.ds(j * LANES, LANES)
                keep = (iot + j * LANES) < ptr
                pval[sl] = jnp.where(keep, pval[sl], 0.0)
            fire(fb, nf)
            drain(1 - fb, nf + 1)
            plsc.subcore_barrier()
            pltpu.sync_copy(acc.at[pl.ds(w * rps, rps)],
                            out_hbm.at[pl.ds(lo + w * rps, rps)])
            plsc.subcore_barrier()

    return spmm(dst, src, vals, table)


def _tc_norm(x):
    r = 1024
    assert x.shape[0] % r == 0

    def body(x_ref, o_ref):
        xb = x_ref[...]
        n = jnp.sqrt(jnp.sum(xb * xb, axis=1, keepdims=True))
        o_ref[...] = xb / jnp.maximum(n, 1e-12)

    return pl.pallas_call(
        body,
        out_shape=jax.ShapeDtypeStruct(x.shape, x.dtype),
        grid=(x.shape[0] // r,),
        in_specs=[pl.BlockSpec((r, D), lambda i: (i, 0))],
        out_specs=pl.BlockSpec((r, D), lambda i: (i, 0)),
    )(x)


def _tc_combine(f0, n1, f2, w3):
    r = 1024
    assert f0.shape[0] % r == 0

    def body(w_ref, f0_ref, n1_ref, f2_ref, o_ref):
        xb = f2_ref[...]
        n = jnp.sqrt(jnp.sum(xb * xb, axis=1, keepdims=True))
        n2 = xb / jnp.maximum(n, 1e-12)
        o_ref[...] = (w_ref[0] * f0_ref[...] + w_ref[1] * n1_ref[...]
                      + w_ref[2] * n2)

    return pl.pallas_call(
        body,
        out_shape=jax.ShapeDtypeStruct(f0.shape, f0.dtype),
        grid=(f0.shape[0] // r,),
        in_specs=[
            pl.BlockSpec(memory_space=pltpu.SMEM),
            pl.BlockSpec((r, D), lambda i: (i, 0)),
            pl.BlockSpec((r, D), lambda i: (i, 0)),
            pl.BlockSpec((r, D), lambda i: (i, 0)),
        ],
        out_specs=pl.BlockSpec((r, D), lambda i: (i, 0)),
    )(w3, f0, n1, f2)


def _tc_fuse(a, b, cc, w3):
    r = 1000
    assert a.shape[0] % r == 0

    def body(w_ref, a_ref, b_ref, c_ref, o_ref):
        o_ref[...] = (w_ref[0] * a_ref[...] + w_ref[1] * b_ref[...]
                      + w_ref[2] * c_ref[...])

    return pl.pallas_call(
        body,
        out_shape=jax.ShapeDtypeStruct(a.shape, a.dtype),
        grid=(a.shape[0] // r,),
        in_specs=[
            pl.BlockSpec(memory_space=pltpu.SMEM),
            pl.BlockSpec((r, D), lambda i: (i, 0)),
            pl.BlockSpec((r, D), lambda i: (i, 0)),
            pl.BlockSpec((r, D), lambda i: (i, 0)),
        ],
        out_specs=pl.BlockSpec((r, D), lambda i: (i, 0)),
    )(w3, a, b, cc)


def _propagate(edges, vals, a_feat, b_feat, layer_coefs):
    na, nbb = a_feat.shape[0], b_feat.shape[0]
    n = na + nbb
    nb, bs = _pick_blocks(n)
    npad = nb * bs
    f0 = jnp.zeros((npad, D), jnp.float32)
    f0 = f0.at[:na].set(a_feat).at[na:n].set(b_feat)
    dst = edges[0].astype(jnp.int32)
    src = edges[1].astype(jnp.int32)
    bd, bsr, bv = _sc_bucketize(dst, src, vals, n_dst=n)
    f1 = _sc_spmm_bkt(bd, bsr, bv, f0, n_dst=n)
    n1 = _tc_norm(f1)
    f2 = _sc_spmm_bkt(bd, bsr, bv, f1, n_dst=n)
    w3 = layer_coefs.reshape(3).astype(jnp.float32)
    comb = _tc_combine(f0, n1, f2, w3)
    return comb, na


def kernel(ub_prop_edges, ub_prop_vals, ui_prop_edges, ui_prop_vals,
           bi_prop_edges, bi_prop_vals, bi_agg_rows, bi_agg_cols, bi_agg_vals,
           ui_agg_rows, ui_agg_cols, ui_agg_vals,
           users_feature, bundles_feature, items_feature,
           modal_coefs, UB_layer_coefs, UI_layer_coefs, BI_layer_coefs):
    nu = users_feature.shape[0]
    nbun = bundles_feature.shape[0]

    ub_comb, _ = _propagate(ub_prop_edges, ub_prop_vals,
                            users_feature, bundles_feature, UB_layer_coefs)
    ui_comb, _ = _propagate(ui_prop_edges, ui_prop_vals,
                            users_feature, items_feature, UI_layer_coefs)
    bi_comb, _ = _propagate(bi_prop_edges, bi_prop_vals,
                            bundles_feature, items_feature, BI_layer_coefs)

    ui_bundles = _sc_spmm(bi_agg_rows.astype(jnp.int32),
                          bi_agg_cols.astype(jnp.int32) + nu,
                          bi_agg_vals, ui_comb, n_dst=nbun)
    bi_users = _sc_spmm(ui_agg_rows.astype(jnp.int32),
                        ui_agg_cols.astype(jnp.int32) + nbun,
                        ui_agg_vals, bi_comb, n_dst=nu)

    mc = modal_coefs.reshape(3).astype(jnp.float32)
    users_rep = _tc_fuse(ub_comb[:nu], ui_comb[:nu], bi_users[:nu], mc)
    bundles_rep = _tc_fuse(ub_comb[nu:nu + nbun], ui_bundles[:nbun],
                           bi_comb[:nbun], mc)
    return users_rep, bundles_rep
